# Initial kernel scaffold; baseline (speedup 1.0000x reference)
import jax
import jax.numpy as jnp
from jax.experimental import pallas as pl


def kernel(x, edge_index, W1, W2):
    raise NotImplementedError("write your pallas kernel here")


# speedup vs baseline: 1.2858x; 1.2858x over previous
import functools
import math

import jax
import jax.numpy as jnp
from jax import lax
from jax.experimental import pallas as pl
from jax.experimental.pallas import tpu as pltpu
from jax.experimental.pallas import tpu_sc as plsc

N = 10000
DEG = 16
D = 256

NW = 32
NP = 320
NPAD = NW * NP
G = 8
ROWS = G * DEG
NG = NP // G
NV = D // 16
INV = 1.0 / math.sqrt(17.0)

BLKR = 1024


def _mm1_body(x_ref, w_ref, o_ref):
    o_ref[...] = jnp.dot(
        x_ref[...], w_ref[...],
        preferred_element_type=jnp.float32,
        precision=lax.Precision.HIGHEST,
    ) * INV


def _mm2_body(s_ref, h_ref, w_ref, o_ref):
    y = (s_ref[...] + h_ref[...]) * INV
    y = jnp.where(y > 0, y, jnp.exp(y) - 1.0)
    o_ref[...] = jnp.dot(
        y, w_ref[...],
        preferred_element_type=jnp.float32,
        precision=lax.Precision.HIGHEST,
    ) * INV


def _elu_body(s_ref, h_ref, o_ref):
    y = (s_ref[...] + h_ref[...]) * INV
    o_ref[...] = jnp.where(y > 0, y, jnp.exp(y) - 1.0)


def _mm1(x, wT):
    return pl.pallas_call(
        _mm1_body,
        grid=(NPAD // BLKR,),
        in_specs=[pl.BlockSpec((BLKR, D), lambda i: (i, 0)),
                  pl.BlockSpec((D, D), lambda i: (0, 0))],
        out_specs=pl.BlockSpec((BLKR, D), lambda i: (i, 0)),
        out_shape=jax.ShapeDtypeStruct((NPAD, D), jnp.float32),
    )(x, wT)


def _mm2(s, h, wT):
    return pl.pallas_call(
        _mm2_body,
        grid=(NPAD // BLKR,),
        in_specs=[pl.BlockSpec((BLKR, D), lambda i: (i, 0)),
                  pl.BlockSpec((BLKR, D), lambda i: (i, 0)),
                  pl.BlockSpec((D, D), lambda i: (0, 0))],
        out_specs=pl.BlockSpec((BLKR, D), lambda i: (i, 0)),
        out_shape=jax.ShapeDtypeStruct((NPAD, D), jnp.float32),
    )(s, h, wT)


def _elu(s, h):
    return pl.pallas_call(
        _elu_body,
        grid=(NPAD // BLKR,),
        in_specs=[pl.BlockSpec((BLKR, D), lambda i: (i, 0)),
                  pl.BlockSpec((BLKR, D), lambda i: (i, 0))],
        out_specs=pl.BlockSpec((BLKR, D), lambda i: (i, 0)),
        out_shape=jax.ShapeDtypeStruct((NPAD, D), jnp.float32),
    )(s, h)


def _gather_sum(h, e_grp):
    mesh = plsc.VectorSubcoreMesh(core_axis_name="c", subcore_axis_name="s")

    @functools.partial(
        pl.kernel,
        mesh=mesh,
        out_type=jax.ShapeDtypeStruct((NPAD, D), jnp.float32),
        scratch_types=[
            pltpu.VMEM((NG, ROWS), jnp.int32),
            pltpu.VMEM((ROWS, D), jnp.float32),
            pltpu.VMEM((ROWS, D), jnp.float32),
            pltpu.VMEM((G, D), jnp.float32),
            pltpu.VMEM((G, D), jnp.float32),
            pltpu.SemaphoreType.DMA,
            pltpu.SemaphoreType.DMA,
            pltpu.SemaphoreType.DMA,
            pltpu.SemaphoreType.DMA,
        ],
    )
    def k(h_hbm, e_hbm, out_hbm, idx_v, rows_a, rows_b, out_a, out_b,
          sem_a, sem_b, sem_oa, sem_ob):
        wid = lax.axis_index("s") * 2 + lax.axis_index("c")
        base = wid * NP
        pltpu.sync_copy(e_hbm.at[wid], idx_v)
        pltpu.async_copy(h_hbm.at[idx_v.at[0]], rows_a, sem_a)
        pltpu.async_copy(h_hbm.at[idx_v.at[1]], rows_b, sem_b)

        def compute(rows_v, out_v):
            def node_body(n, carry):
                r0 = n * DEG
                for v in range(NV):
                    sl = pl.ds(v * 16, 16)
                    acc = rows_v[r0, sl]
                    for j in range(1, DEG):
                        acc = acc + rows_v[r0 + j, sl]
                    out_v[n, sl] = acc
                return carry
            lax.fori_loop(0, G, node_body, 0)

        def step(g, rows_v, out_v, sem_g, sem_o):
            pltpu.make_async_copy(h_hbm.at[idx_v.at[g]], rows_v, sem_g).wait()

            @pl.when(g >= 2)
            def _():
                pltpu.make_async_copy(
                    out_v, out_hbm.at[pl.ds(base, G)], sem_o).wait()

            compute(rows_v, out_v)
            pltpu.async_copy(out_v, out_hbm.at[pl.ds(base + g * G, G)], sem_o)

            @pl.when(g + 2 < NG)
            def _():
                pltpu.async_copy(h_hbm.at[idx_v.at[g + 2]], rows_v, sem_g)

        def pair(p, carry):
            step(2 * p, rows_a, out_a, sem_a, sem_oa)
            step(2 * p + 1, rows_b, out_b, sem_b, sem_ob)
            return carry

        lax.fori_loop(0, NG // 2, pair, 0)
        pltpu.make_async_copy(out_a, out_hbm.at[pl.ds(base, G)], sem_oa).wait()
        pltpu.make_async_copy(out_b, out_hbm.at[pl.ds(base, G)], sem_ob).wait()

    return k(h, e_grp)


def kernel(x, edge_index, W1, W2):
    x_pad = jnp.zeros((NPAD, D), jnp.float32).at[:N].set(x)
    e = edge_index.astype(jnp.int32)
    e_grp = jnp.zeros((NPAD, DEG), jnp.int32).at[:N].set(e).reshape(NW, NG, ROWS)
    w1T = W1.T
    w2T = W2.T

    h1 = _mm1(x_pad, w1T)
    s1 = _gather_sum(h1, e_grp)
    h2 = _mm2(s1, h1, w2T)
    s2 = _gather_sum(h2, e_grp)
    out = _elu(s2, h2)
    return out[:N]


# speedup vs baseline: 1.2950x; 1.0071x over previous
import functools
import math

import jax
import jax.numpy as jnp
from jax import lax
from jax.experimental import pallas as pl
from jax.experimental.pallas import tpu as pltpu
from jax.experimental.pallas import tpu_sc as plsc

N = 10000
DEG = 16
D = 256

NW = 32
NP = 320
NPAD = NW * NP
G = 8
ROWS = G * DEG
NG = NP // G
NV = D // 16
INV = 1.0 / math.sqrt(17.0)

BLKR = 1024


def _mm1_body(x_ref, w_ref, o_ref):
    o_ref[...] = jnp.dot(
        x_ref[...], w_ref[...],
        preferred_element_type=jnp.float32,
        precision=lax.Precision.HIGHEST,
    ) * INV


def _mm2_body(s_ref, h_ref, w_ref, o_ref):
    y = (s_ref[...] + h_ref[...]) * INV
    y = jnp.where(y > 0, y, jnp.exp(y) - 1.0)
    o_ref[...] = jnp.dot(
        y, w_ref[...],
        preferred_element_type=jnp.float32,
        precision=lax.Precision.HIGHEST,
    ) * INV


def _elu_body(s_ref, h_ref, o_ref):
    y = (s_ref[...] + h_ref[...]) * INV
    o_ref[...] = jnp.where(y > 0, y, jnp.exp(y) - 1.0)


def _mm1(x, wT):
    return pl.pallas_call(
        _mm1_body,
        grid=(NPAD // BLKR,),
        in_specs=[pl.BlockSpec((BLKR, D), lambda i: (i, 0)),
                  pl.BlockSpec((D, D), lambda i: (0, 0))],
        out_specs=pl.BlockSpec((BLKR, D), lambda i: (i, 0)),
        out_shape=jax.ShapeDtypeStruct((NPAD, D), jnp.float32),
    )(x, wT)


def _mm2(s, h, wT):
    return pl.pallas_call(
        _mm2_body,
        grid=(NPAD // BLKR,),
        in_specs=[pl.BlockSpec((BLKR, D), lambda i: (i, 0)),
                  pl.BlockSpec((BLKR, D), lambda i: (i, 0)),
                  pl.BlockSpec((D, D), lambda i: (0, 0))],
        out_specs=pl.BlockSpec((BLKR, D), lambda i: (i, 0)),
        out_shape=jax.ShapeDtypeStruct((NPAD, D), jnp.float32),
    )(s, h, wT)


def _elu(s, h):
    return pl.pallas_call(
        _elu_body,
        grid=(NPAD // BLKR,),
        in_specs=[pl.BlockSpec((BLKR, D), lambda i: (i, 0)),
                  pl.BlockSpec((BLKR, D), lambda i: (i, 0))],
        out_specs=pl.BlockSpec((BLKR, D), lambda i: (i, 0)),
        out_shape=jax.ShapeDtypeStruct((NPAD, D), jnp.float32),
    )(s, h)


def _gather_sum(h, e_grp):
    mesh = plsc.VectorSubcoreMesh(core_axis_name="c", subcore_axis_name="s")

    @functools.partial(
        pl.kernel,
        mesh=mesh,
        out_type=jax.ShapeDtypeStruct((NPAD, D), jnp.float32),
        scratch_types=[
            pltpu.VMEM((NG, ROWS), jnp.int32),
            pltpu.VMEM((ROWS, D), jnp.float32),
            pltpu.VMEM((ROWS, D), jnp.float32),
            pltpu.VMEM((G, D), jnp.float32),
            pltpu.VMEM((G, D), jnp.float32),
            pltpu.SemaphoreType.DMA,
            pltpu.SemaphoreType.DMA,
            pltpu.SemaphoreType.DMA,
            pltpu.SemaphoreType.DMA,
        ],
    )
    def k(h_hbm, e_hbm, out_hbm, idx_v, rows_a, rows_b, out_a, out_b,
          sem_a, sem_b, sem_oa, sem_ob):
        wid = lax.axis_index("s") * 2 + lax.axis_index("c")
        base = wid * NP
        pltpu.sync_copy(e_hbm.at[wid], idx_v)
        pltpu.async_copy(h_hbm.at[idx_v.at[0]], rows_a, sem_a)
        pltpu.async_copy(h_hbm.at[idx_v.at[1]], rows_b, sem_b)

        def compute(rows_v, out_v):
            def node_body(n, carry):
                r0 = n * DEG
                for v in range(NV):
                    sl = pl.ds(v * 16, 16)
                    vals = [rows_v[r0 + j, sl] for j in range(DEG)]
                    while len(vals) > 1:
                        nxt = [vals[i] + vals[i + 1]
                               for i in range(0, len(vals) - 1, 2)]
                        if len(vals) % 2:
                            nxt.append(vals[-1])
                        vals = nxt
                    out_v[n, sl] = vals[0]
                return carry
            lax.fori_loop(0, G, node_body, 0)

        def step(g, rows_v, out_v, sem_g, sem_o):
            pltpu.make_async_copy(h_hbm.at[idx_v.at[g]], rows_v, sem_g).wait()

            @pl.when(g >= 2)
            def _():
                pltpu.make_async_copy(
                    out_v, out_hbm.at[pl.ds(base, G)], sem_o).wait()

            compute(rows_v, out_v)
            pltpu.async_copy(out_v, out_hbm.at[pl.ds(base + g * G, G)], sem_o)

            @pl.when(g + 2 < NG)
            def _():
                pltpu.async_copy(h_hbm.at[idx_v.at[g + 2]], rows_v, sem_g)

        def pair(p, carry):
            step(2 * p, rows_a, out_a, sem_a, sem_oa)
            step(2 * p + 1, rows_b, out_b, sem_b, sem_ob)
            return carry

        lax.fori_loop(0, NG // 2, pair, 0)
        pltpu.make_async_copy(out_a, out_hbm.at[pl.ds(base, G)], sem_oa).wait()
        pltpu.make_async_copy(out_b, out_hbm.at[pl.ds(base, G)], sem_ob).wait()

    return k(h, e_grp)


def kernel(x, edge_index, W1, W2):
    x_pad = jnp.zeros((NPAD, D), jnp.float32).at[:N].set(x)
    e = edge_index.astype(jnp.int32)
    e_grp = jnp.zeros((NPAD, DEG), jnp.int32).at[:N].set(e).reshape(NW, NG, ROWS)
    w1T = W1.T
    w2T = W2.T

    h1 = _mm1(x_pad, w1T)
    s1 = _gather_sum(h1, e_grp)
    h2 = _mm2(s1, h1, w2T)
    s2 = _gather_sum(h2, e_grp)
    out = _elu(s2, h2)
    return out[:N]


# speedup vs baseline: 1.3509x; 1.0432x over previous
import functools
import math

import numpy as np

import jax
import jax.numpy as jnp
from jax import lax
from jax.experimental import pallas as pl
from jax.experimental.pallas import tpu as pltpu
from jax.experimental.pallas import tpu_sc as plsc

N = 10000
DEG = 16
D = 256

NW = 32
NP = 320
NPAD = NW * NP
G = 8
ROWS = G * DEG
NG = NP // G
NV = D // 16
INV = 1.0 / math.sqrt(17.0)

BLKR = 1024


def _mm1_body(x_ref, w_ref, wb_ref, o_ref, ob_ref):
    h = jnp.dot(
        x_ref[...], w_ref[...],
        preferred_element_type=jnp.float32,
        precision=lax.Precision.HIGHEST,
    ) * INV
    o_ref[...] = h
    hb = jnp.dot(
        x_ref[...].astype(jnp.bfloat16), wb_ref[...],
        preferred_element_type=jnp.float32,
    ) * INV
    ob_ref[...] = hb.astype(jnp.bfloat16)


def _mm2_body(s_ref, h_ref, w_ref, wb_ref, o_ref, ob_ref):
    y = (s_ref[...] + h_ref[...]) * INV
    y = jnp.where(y > 0, y, jnp.exp(y) - 1.0)
    o_ref[...] = jnp.dot(
        y, w_ref[...],
        preferred_element_type=jnp.float32,
        precision=lax.Precision.HIGHEST,
    ) * INV
    hb = jnp.dot(
        y.astype(jnp.bfloat16), wb_ref[...],
        preferred_element_type=jnp.float32,
    ) * INV
    ob_ref[...] = hb.astype(jnp.bfloat16)


def _elu_body(s_ref, h_ref, o_ref):
    y = (s_ref[...] + h_ref[...]) * INV
    o_ref[...] = jnp.where(y > 0, y, jnp.exp(y) - 1.0)


def _mm1(x, wT, wTb):
    return pl.pallas_call(
        _mm1_body,
        grid=(NPAD // BLKR,),
        in_specs=[pl.BlockSpec((BLKR, D), lambda i: (i, 0)),
                  pl.BlockSpec((D, D), lambda i: (0, 0)),
                  pl.BlockSpec((D, D), lambda i: (0, 0))],
        out_specs=[pl.BlockSpec((BLKR, D), lambda i: (i, 0)),
                   pl.BlockSpec((BLKR, D), lambda i: (i, 0))],
        out_shape=[jax.ShapeDtypeStruct((NPAD, D), jnp.float32),
                   jax.ShapeDtypeStruct((NPAD, D), jnp.bfloat16)],
    )(x, wT, wTb)


def _mm2(s, h, wT, wTb):
    return pl.pallas_call(
        _mm2_body,
        grid=(NPAD // BLKR,),
        in_specs=[pl.BlockSpec((BLKR, D), lambda i: (i, 0)),
                  pl.BlockSpec((BLKR, D), lambda i: (i, 0)),
                  pl.BlockSpec((D, D), lambda i: (0, 0)),
                  pl.BlockSpec((D, D), lambda i: (0, 0))],
        out_specs=[pl.BlockSpec((BLKR, D), lambda i: (i, 0)),
                   pl.BlockSpec((BLKR, D), lambda i: (i, 0))],
        out_shape=[jax.ShapeDtypeStruct((NPAD, D), jnp.float32),
                   jax.ShapeDtypeStruct((NPAD, D), jnp.bfloat16)],
    )(s, h, wT, wTb)


def _elu(s, h):
    return pl.pallas_call(
        _elu_body,
        grid=(NPAD // BLKR,),
        in_specs=[pl.BlockSpec((BLKR, D), lambda i: (i, 0)),
                  pl.BlockSpec((BLKR, D), lambda i: (i, 0))],
        out_specs=pl.BlockSpec((BLKR, D), lambda i: (i, 0)),
        out_shape=jax.ShapeDtypeStruct((NPAD, D), jnp.float32),
    )(s, h)


def _gather_sum(hb, e_grp):
    mesh = plsc.VectorSubcoreMesh(core_axis_name="c", subcore_axis_name="s")

    @functools.partial(
        pl.kernel,
        mesh=mesh,
        out_type=jax.ShapeDtypeStruct((NPAD, D), jnp.float32),
        scratch_types=[
            pltpu.VMEM((NG, ROWS), jnp.int32),
            pltpu.VMEM((ROWS, D // 2), jnp.int32),
            pltpu.VMEM((ROWS, D // 2), jnp.int32),
            pltpu.VMEM((G, D), jnp.float32),
            pltpu.VMEM((G, D), jnp.float32),
            pltpu.SemaphoreType.DMA,
            pltpu.SemaphoreType.DMA,
            pltpu.SemaphoreType.DMA,
            pltpu.SemaphoreType.DMA,
        ],
    )
    def k(h_hbm, e_hbm, out_hbm, idx_v, rows_a, rows_b, out_a, out_b,
          sem_a, sem_b, sem_oa, sem_ob):
        wid = lax.axis_index("s") * 2 + lax.axis_index("c")
        base = wid * NP
        pltpu.sync_copy(e_hbm.at[wid], idx_v)
        pltpu.async_copy(h_hbm.at[idx_v.at[0]], rows_a, sem_a)
        pltpu.async_copy(h_hbm.at[idx_v.at[1]], rows_b, sem_b)

        def compute(rows_v, out_v):
            def tree_sum(vals):
                while len(vals) > 1:
                    nxt = [vals[i] + vals[i + 1]
                           for i in range(0, len(vals) - 1, 2)]
                    if len(vals) % 2:
                        nxt.append(vals[-1])
                    vals = nxt
                return vals[0]

            NKK = D // 32

            def loadw(r0, kk):
                sl = pl.ds(kk * 16, 16)
                return [rows_v[r0 + j, sl] for j in range(DEG)]

            def node_body(n, carry):
                r0 = n * DEG
                ws = loadw(r0, 0)
                for kk in range(NKK):
                    ws_next = loadw(r0, kk + 1) if kk + 1 < NKK else None
                    los = [lax.bitcast_convert_type(w << 16, jnp.float32)
                           for w in ws]
                    his = [lax.bitcast_convert_type(w & jnp.int32(-65536),
                                                    jnp.float32) for w in ws]
                    out_v[n, pl.ds(kk * 16, 16)] = tree_sum(los)
                    out_v[n, pl.ds(128 + kk * 16, 16)] = tree_sum(his)
                    ws = ws_next
                return carry
            lax.fori_loop(0, G, node_body, 0)

        def step(g, rows_v, out_v, sem_g, sem_o):
            pltpu.make_async_copy(h_hbm.at[idx_v.at[g]], rows_v, sem_g).wait()

            @pl.when(g >= 2)
            def _():
                pltpu.make_async_copy(
                    out_v, out_hbm.at[pl.ds(base, G)], sem_o).wait()

            compute(rows_v, out_v)
            pltpu.async_copy(out_v, out_hbm.at[pl.ds(base + g * G, G)], sem_o)

            @pl.when(g + 2 < NG)
            def _():
                pltpu.async_copy(h_hbm.at[idx_v.at[g + 2]], rows_v, sem_g)

        def pair(p, carry):
            step(2 * p, rows_a, out_a, sem_a, sem_oa)
            step(2 * p + 1, rows_b, out_b, sem_b, sem_ob)
            return carry

        lax.fori_loop(0, NG // 2, pair, 0)
        pltpu.make_async_copy(out_a, out_hbm.at[pl.ds(base, G)], sem_oa).wait()
        pltpu.make_async_copy(out_b, out_hbm.at[pl.ds(base, G)], sem_ob).wait()

    return k(hb, e_grp)


_PI = np.empty((D,), dtype=np.int32)
for _k in range(D // 32):
    for _m in range(16):
        _PI[32 * _k + 2 * _m] = 16 * _k + _m
        _PI[32 * _k + 2 * _m + 1] = 128 + 16 * _k + _m


def kernel(x, edge_index, W1, W2):
    x_pad = jnp.zeros((NPAD, D), jnp.float32).at[:N].set(x)
    e = edge_index.astype(jnp.int32)
    e_grp = jnp.zeros((NPAD, DEG), jnp.int32).at[:N].set(e).reshape(NW, NG, ROWS)
    w1T = W1.T
    w2T = W2.T
    pi = jnp.asarray(_PI)
    w1Tb = w1T[:, pi].astype(jnp.bfloat16)
    w2Tb = w2T[:, pi].astype(jnp.bfloat16)

    def _pairs(hb):
        return jax.lax.bitcast_convert_type(
            hb.reshape(NPAD, D // 2, 2), jnp.int32)

    h1, hb1 = _mm1(x_pad, w1T, w1Tb)
    s1 = _gather_sum(_pairs(hb1), e_grp)
    h2, hb2 = _mm2(s1, h1, w2T, w2Tb)
    s2 = _gather_sum(_pairs(hb2), e_grp)
    out = _elu(s2, h2)
    return out[:N]
